# baseline (speedup 1.0000x reference)
import functools

import jax
import jax.numpy as jnp
from jax import lax
from jax.experimental import pallas as pl
from jax.experimental.pallas import tpu as pltpu
from jax.experimental.pallas import tpu_sc as plsc

VOCAB = 1_000_000
H = 64
B = 16384
L = 20

NC = 2
NS = 16
NW = NC * NS
BPW = B // NW
CB = 32
NCH = BPW // CB
RPC = CB * L
NG = RPC // 128


def _sc_scores_body(center_hbm, ctx_hbm, u_hbm, v_hbm, out_hbm,
                    cidx, vidx, urows, vrows, sbuf, gsem):
    wid = lax.axis_index("s") * NC + lax.axis_index("c")

    def fire(g, slot):
        base = pl.multiple_of(wid * BPW + g * CB, CB)
        pltpu.sync_copy(center_hbm.at[pl.ds(base, CB)], cidx.at[slot])
        off = pl.multiple_of((wid * BPW + g * CB) * L, RPC)
        pltpu.sync_copy(ctx_hbm.at[pl.ds(off, RPC)], vidx.at[slot])
        pltpu.async_copy(u_hbm.at[cidx.at[slot]], urows.at[slot],
                         gsem.at[slot])
        for j in range(NG):
            pltpu.async_copy(v_hbm.at[vidx.at[slot, pl.ds(j * 128, 128)]],
                             vrows.at[slot, pl.ds(j * 128, 128)],
                             gsem.at[slot])

    def wait_gathers(slot):
        pltpu.make_async_copy(u_hbm.at[cidx.at[slot]], urows.at[slot],
                              gsem.at[slot]).wait()
        for j in range(NG):
            pltpu.make_async_copy(v_hbm.at[vidx.at[slot, pl.ds(j * 128, 128)]],
                                  vrows.at[slot, pl.ds(j * 128, 128)],
                                  gsem.at[slot]).wait()

    def compute(g, slot):
        iota16 = lax.iota(jnp.int32, 16)

        def bbody(b, carry):
            r_a = b * L + iota16
            r_b = r_a + 16
            acc_a = jnp.zeros((16,), jnp.float32)
            acc_b = jnp.zeros((16,), jnp.float32)
            uvecs = [urows[slot, b, pl.ds(q * 16, 16)] for q in range(H // 16)]
            for h in range(H):
                col = jnp.full((16,), h, jnp.int32)
                g_a = plsc.load_gather(vrows.at[slot], [r_a, col])
                g_b = plsc.load_gather(vrows.at[slot], [r_b, col])
                uv = uvecs[h // 16][h % 16]
                acc_a = acc_a + uv * g_a
                acc_b = acc_b + uv * g_b
            sbuf[slot, pl.ds(b * L, 16)] = acc_a
            sbuf[slot, pl.ds(b * L + 16, 16)] = acc_b
            return carry

        lax.fori_loop(0, CB, bbody, 0)
        base = pl.multiple_of((wid * BPW + g * CB) * L, RPC)
        pltpu.sync_copy(sbuf.at[slot, pl.ds(0, RPC)],
                        out_hbm.at[pl.ds(base, RPC)])

    fire(0, 0)

    def pair(i, carry):
        for s in (0, 1):
            g = i * 2 + s

            @pl.when(g + 1 < NCH)
            def _():
                fire(g + 1, (s + 1) % 2)

            wait_gathers(s)
            compute(g, s)
        return carry

    lax.fori_loop(0, NCH // 2, pair, 0)


def _sc_scores(center_flat, ctx_flat, u, v):
    mesh = plsc.VectorSubcoreMesh(core_axis_name="c", subcore_axis_name="s",
                                  num_cores=NC, num_subcores=NS)
    return pl.kernel(
        _sc_scores_body,
        out_type=jax.ShapeDtypeStruct((B * L,), jnp.float32),
        mesh=mesh,
        scratch_types=[
            pltpu.VMEM((2, CB), jnp.int32),
            pltpu.VMEM((2, RPC), jnp.int32),
            pltpu.VMEM((2, CB, H), jnp.float32),
            pltpu.VMEM((2, RPC + 16, H), jnp.float32),
            pltpu.VMEM((2, RPC + 32), jnp.float32),
            pltpu.SemaphoreType.DMA((2,)),
        ],
        compiler_params=pltpu.CompilerParams(needs_layout_passes=False,
                                             use_tc_tiling_on_sc=False),
    )(center_flat, ctx_flat, u, v)


def _loss_body(s_ref, lab_ref, m_ref, out_ref):
    s = s_ref[...]
    lab = lab_ref[...]
    m = m_ref[...]
    per = jnp.maximum(s, 0.0) - s * lab + jnp.log1p(jnp.exp(-jnp.abs(s)))
    num = jnp.sum(per * m)
    den = jnp.maximum(jnp.sum(m), 1.0)
    out_ref[0, 0] = num / den


def _tc_loss(scores2d, label2d, mask2d):
    return pl.pallas_call(
        _loss_body,
        out_shape=jax.ShapeDtypeStruct((1, 1), jnp.float32),
        out_specs=pl.BlockSpec(memory_space=pltpu.SMEM),
    )(scores2d, label2d, mask2d)


def kernel(center, context_neg, label, mask, U, V):
    center_flat = center.reshape(B)
    ctx_flat = context_neg.reshape(B * L)
    scores = _sc_scores(center_flat, ctx_flat, U, V)
    scores2d = scores.reshape(B * L // 128, 128)
    label2d = label.reshape(B * L // 128, 128)
    mask2d = mask.reshape(B * L // 128, 128)
    return _tc_loss(scores2d, label2d, mask2d).reshape(())
